# Initial kernel scaffold; baseline (speedup 1.0000x reference)
import jax
import jax.numpy as jnp
from jax.experimental import pallas as pl


def kernel(input_embeddings, pos_table):
    raise NotImplementedError("write your pallas kernel here")


# speedup vs baseline: 1.4952x; 1.4952x over previous
import jax
import jax.numpy as jnp
from jax.experimental import pallas as pl


def _add_body(x_ref, p_ref, o_ref):
    o_ref[...] = x_ref[...] + p_ref[...]


def kernel(input_embeddings, pos_table):
    B, S, D = input_embeddings.shape
    TS = 512
    grid = (S // TS, B)
    return pl.pallas_call(
        _add_body,
        grid=grid,
        in_specs=[
            pl.BlockSpec((1, TS, D), lambda s, b: (b, s, 0)),
            pl.BlockSpec((TS, D), lambda s, b: (s, 0)),
        ],
        out_specs=pl.BlockSpec((1, TS, D), lambda s, b: (b, s, 0)),
        out_shape=jax.ShapeDtypeStruct((B, S, D), input_embeddings.dtype),
    )(input_embeddings, pos_table)
